# baseline (device time: 105213 ns/iter reference)
import jax
import jax.numpy as jnp
from jax import lax
from jax.experimental import pallas as pl
from jax.experimental.pallas import tpu as pltpu

N_DEV = 8


def kernel(x, w_mat):
    m_per, k = x.shape
    _, n_per = w_mat.shape

    def body(x_ref, w_ref, out_ref, gathered, w_bf, send_sems, recv_sems):
        my = lax.axis_index("i")
        left = lax.rem(my + N_DEV - 1, N_DEV)
        right = lax.rem(my + 1, N_DEV)

        barrier_sem = pltpu.get_barrier_semaphore()
        for nbr in (left, right):
            pl.semaphore_signal(
                barrier_sem, inc=1,
                device_id=(nbr,), device_id_type=pl.DeviceIdType.MESH,
            )
        pl.semaphore_wait(barrier_sem, 2)

        x_bf = x_ref[:, :].astype(jnp.bfloat16)
        w_bf[:, :] = w_ref[:, :].astype(jnp.bfloat16)
        gathered[pl.ds(my * m_per, m_per), :] = x_bf

        out_ref[pl.ds(my * m_per, m_per), :] = jnp.maximum(
            jnp.dot(x_bf, w_bf[:, :], preferred_element_type=jnp.float32), 0.0
        )

        for h in range(N_DEV - 1):
            o_send = lax.rem(my - h + N_DEV, N_DEV)
            o_recv = lax.rem(my - h - 1 + N_DEV, N_DEV)
            rdma = pltpu.make_async_remote_copy(
                src_ref=gathered.at[pl.ds(o_send * m_per, m_per), :],
                dst_ref=gathered.at[pl.ds(o_send * m_per, m_per), :],
                send_sem=send_sems.at[h],
                recv_sem=recv_sems.at[h],
                device_id=(right,),
                device_id_type=pl.DeviceIdType.MESH,
            )
            rdma.start()
            rdma.wait()
            out_ref[pl.ds(o_recv * m_per, m_per), :] = jnp.maximum(
                jnp.dot(
                    gathered[pl.ds(o_recv * m_per, m_per), :],
                    w_bf[:, :],
                    preferred_element_type=jnp.float32,
                ),
                0.0,
            )

    return pl.pallas_call(
        body,
        out_shape=jax.ShapeDtypeStruct((N_DEV * m_per, n_per), jnp.float32),
        in_specs=[
            pl.BlockSpec(memory_space=pltpu.VMEM),
            pl.BlockSpec(memory_space=pltpu.VMEM),
        ],
        out_specs=pl.BlockSpec(memory_space=pltpu.VMEM),
        scratch_shapes=[
            pltpu.VMEM((N_DEV * m_per, k), jnp.bfloat16),
            pltpu.VMEM((k, n_per), jnp.bfloat16),
            pltpu.SemaphoreType.DMA((N_DEV - 1,)),
            pltpu.SemaphoreType.DMA((N_DEV - 1,)),
        ],
        compiler_params=pltpu.CompilerParams(collective_id=0),
    )(x, w_mat)


# device time: 55761 ns/iter; 1.8869x vs baseline; 1.8869x over previous
import jax
import jax.numpy as jnp
from jax import lax
from jax.experimental import pallas as pl
from jax.experimental.pallas import tpu as pltpu

N_DEV = 8
N_HOP = 4


def kernel(x, w_mat):
    m_per, k = x.shape
    _, n_per = w_mat.shape
    half = m_per // 2

    def body(x_ref, w_ref, out_ref, gathered, w_bf, cw_send, cw_recv,
             ccw_send, ccw_recv):
        my = lax.axis_index("i")
        left = lax.rem(my + N_DEV - 1, N_DEV)
        right = lax.rem(my + 1, N_DEV)

        barrier_sem = pltpu.get_barrier_semaphore()
        for nbr in (left, right):
            pl.semaphore_signal(
                barrier_sem, inc=1,
                device_id=(nbr,), device_id_type=pl.DeviceIdType.MESH,
            )
        pl.semaphore_wait(barrier_sem, 2)

        x_bf = x_ref[:, :].astype(jnp.bfloat16)
        gathered[pl.ds(my * m_per, m_per), :] = x_bf

        def chunk_slice(o, direction, h):
            if h < N_HOP - 1:
                return pl.ds(o * m_per, m_per)
            if direction == "cw":
                return pl.ds(o * m_per, half)
            return pl.ds(o * m_per + half, half)

        def mk(direction, h):
            if direction == "cw":
                o = lax.rem(my - h + N_DEV, N_DEV)
                dst_dev, sends, recvs = right, cw_send, cw_recv
            else:
                o = lax.rem(my + h, N_DEV)
                dst_dev, sends, recvs = left, ccw_send, ccw_recv
            sl = chunk_slice(o, direction, h)
            return pltpu.make_async_remote_copy(
                src_ref=gathered.at[sl, :],
                dst_ref=gathered.at[sl, :],
                send_sem=sends.at[h],
                recv_sem=recvs.at[h],
                device_id=(dst_dev,),
                device_id_type=pl.DeviceIdType.MESH,
            )

        rdmas = {}
        rdmas["cw", 0] = mk("cw", 0)
        rdmas["cw", 0].start()
        rdmas["ccw", 0] = mk("ccw", 0)
        rdmas["ccw", 0].start()

        w_bf[:, :] = w_ref[:, :].astype(jnp.bfloat16)

        def gemm(row_start, rows):
            out_ref[pl.ds(row_start, rows), :] = jnp.maximum(
                jnp.dot(
                    gathered[pl.ds(row_start, rows), :],
                    w_bf[:, :],
                    preferred_element_type=jnp.float32,
                ),
                0.0,
            )

        gemm_own_start = my * m_per
        out_ref[pl.ds(gemm_own_start, m_per), :] = jnp.maximum(
            jnp.dot(x_bf, w_bf[:, :], preferred_element_type=jnp.float32), 0.0
        )

        for h in range(N_HOP):
            o_cw = lax.rem(my - h - 1 + N_DEV, N_DEV)
            rdmas["cw", h].wait_recv()
            if h + 1 < N_HOP:
                rdmas["cw", h + 1] = mk("cw", h + 1)
                rdmas["cw", h + 1].start()
            o_ccw = lax.rem(my + h + 1, N_DEV)
            rdmas["ccw", h].wait_recv()
            if h + 1 < N_HOP:
                rdmas["ccw", h + 1] = mk("ccw", h + 1)
                rdmas["ccw", h + 1].start()
            if h < N_HOP - 1:
                gemm(o_cw * m_per, m_per)
                gemm(o_ccw * m_per, m_per)
            else:
                gemm(o_cw * m_per, half)
                gemm(o_ccw * m_per + half, half)

        for h in range(N_HOP):
            rdmas["cw", h].wait_send()
            rdmas["ccw", h].wait_send()

    return pl.pallas_call(
        body,
        out_shape=jax.ShapeDtypeStruct((N_DEV * m_per, n_per), jnp.float32),
        in_specs=[
            pl.BlockSpec(memory_space=pltpu.VMEM),
            pl.BlockSpec(memory_space=pltpu.VMEM),
        ],
        out_specs=pl.BlockSpec(memory_space=pltpu.VMEM),
        scratch_shapes=[
            pltpu.VMEM((N_DEV * m_per, k), jnp.bfloat16),
            pltpu.VMEM((k, n_per), jnp.bfloat16),
            pltpu.SemaphoreType.DMA((N_HOP,)),
            pltpu.SemaphoreType.DMA((N_HOP,)),
            pltpu.SemaphoreType.DMA((N_HOP,)),
            pltpu.SemaphoreType.DMA((N_HOP,)),
        ],
        compiler_params=pltpu.CompilerParams(collective_id=0),
    )(x, w_mat)


# device time: 48600 ns/iter; 2.1649x vs baseline; 1.1473x over previous
import jax
import jax.numpy as jnp
from jax import lax
from jax.experimental import pallas as pl
from jax.experimental.pallas import tpu as pltpu

N_DEV = 8
N_HOP = 3


def kernel(x, w_mat):
    m_per, k = x.shape
    _, n_per = w_mat.shape

    def body(x_ref, w_ref, out_ref, gathered, w_bf, cw_send, cw_recv,
             ccw_send, ccw_recv, ch_send, ch_recv):
        me = lax.axis_index("i")

        def g(r):
            return jnp.where(r < 4, r, 11 - r)

        ri = g(me)
        right_log = g(lax.rem(ri + 1, N_DEV))
        left_log = g(lax.rem(ri + 7, N_DEV))
        even = lax.rem(ri, 2) == 0
        partner_log = g(lax.rem(jnp.where(even, ri + 3, ri + 5), N_DEV))
        fwd_log = g(lax.rem(jnp.where(even, ri + 7, ri + 1), N_DEV))

        barrier_sem = pltpu.get_barrier_semaphore()
        for nbr in (left_log, right_log, partner_log):
            pl.semaphore_signal(
                barrier_sem, inc=1,
                device_id=(nbr,), device_id_type=pl.DeviceIdType.MESH,
            )
        pl.semaphore_wait(barrier_sem, 3)

        x_bf = x_ref[:, :].astype(jnp.bfloat16)
        gathered[pl.ds(me * m_per, m_per), :] = x_bf

        def mk_ring(direction, h):
            if direction == "cw":
                o = g(lax.rem(ri - h + N_DEV, N_DEV))
                dst_dev, sends, recvs = right_log, cw_send, cw_recv
            else:
                o = g(lax.rem(ri + h, N_DEV))
                dst_dev, sends, recvs = left_log, ccw_send, ccw_recv
            sl = pl.ds(o * m_per, m_per)
            return pltpu.make_async_remote_copy(
                src_ref=gathered.at[sl, :],
                dst_ref=gathered.at[sl, :],
                send_sem=sends.at[h],
                recv_sem=recvs.at[h],
                device_id=(dst_dev,),
                device_id_type=pl.DeviceIdType.MESH,
            )

        rd = {}
        rd["cw", 0] = mk_ring("cw", 0)
        rd["cw", 0].start()
        rd["ccw", 0] = mk_ring("ccw", 0)
        rd["ccw", 0].start()

        w_bf[:, :] = w_ref[:, :].astype(jnp.bfloat16)

        def gemm(o_log, x_block=None):
            sl = pl.ds(o_log * m_per, m_per)
            a = gathered[sl, :] if x_block is None else x_block
            out_ref[sl, :] = jnp.maximum(
                jnp.dot(a, w_bf[:, :], preferred_element_type=jnp.float32),
                0.0,
            )

        gemm(me, x_bf)

        rd["cw", 0].wait_recv()
        rd["cw", 1] = mk_ring("cw", 1)
        rd["cw", 1].start()
        rd["ccw", 0].wait_recv()
        rd["ccw", 1] = mk_ring("ccw", 1)
        rd["ccw", 1].start()
        ch_sl = pl.ds(fwd_log * m_per, m_per)
        chord = pltpu.make_async_remote_copy(
            src_ref=gathered.at[ch_sl, :],
            dst_ref=gathered.at[ch_sl, :],
            send_sem=ch_send.at[0],
            recv_sem=ch_recv.at[0],
            device_id=(partner_log,),
            device_id_type=pl.DeviceIdType.MESH,
        )
        chord.start()
        gemm(g(lax.rem(ri + 7, N_DEV)))
        gemm(g(lax.rem(ri + 1, N_DEV)))

        rd["cw", 1].wait_recv()
        rd["cw", 2] = mk_ring("cw", 2)
        rd["cw", 2].start()
        rd["ccw", 1].wait_recv()
        rd["ccw", 2] = mk_ring("ccw", 2)
        rd["ccw", 2].start()
        gemm(g(lax.rem(ri + 6, N_DEV)))
        gemm(g(lax.rem(ri + 2, N_DEV)))

        chord.wait_recv()
        gemm(g(lax.rem(ri + 4, N_DEV)))

        rd["cw", 2].wait_recv()
        gemm(g(lax.rem(ri + 5, N_DEV)))
        rd["ccw", 2].wait_recv()
        gemm(g(lax.rem(ri + 3, N_DEV)))

        for h in range(N_HOP):
            rd["cw", h].wait_send()
            rd["ccw", h].wait_send()
        chord.wait_send()

    return pl.pallas_call(
        body,
        out_shape=jax.ShapeDtypeStruct((N_DEV * m_per, n_per), jnp.float32),
        in_specs=[
            pl.BlockSpec(memory_space=pltpu.VMEM),
            pl.BlockSpec(memory_space=pltpu.VMEM),
        ],
        out_specs=pl.BlockSpec(memory_space=pltpu.VMEM),
        scratch_shapes=[
            pltpu.VMEM((N_DEV * m_per, k), jnp.bfloat16),
            pltpu.VMEM((k, n_per), jnp.bfloat16),
            pltpu.SemaphoreType.DMA((N_HOP,)),
            pltpu.SemaphoreType.DMA((N_HOP,)),
            pltpu.SemaphoreType.DMA((N_HOP,)),
            pltpu.SemaphoreType.DMA((N_HOP,)),
            pltpu.SemaphoreType.DMA((1,)),
            pltpu.SemaphoreType.DMA((1,)),
        ],
        compiler_params=pltpu.CompilerParams(collective_id=0),
    )(x, w_mat)


# device time: 44801 ns/iter; 2.3485x vs baseline; 1.0848x over previous
import jax
import jax.numpy as jnp
from jax import lax
from jax.experimental import pallas as pl
from jax.experimental.pallas import tpu as pltpu

N_DEV = 8
N_HOP = 3
N_PC = 2


def kernel(x, w_mat):
    m_per, k = x.shape
    _, n_per = w_mat.shape
    half = m_per // N_PC

    def body(x_ref, w_ref, out_ref, gathered, w_bf, cw_send, cw_recv,
             ccw_send, ccw_recv, ch_send, ch_recv):
        me = lax.axis_index("i")

        def g(r):
            return jnp.where(r < 4, r, 11 - r)

        ri = g(me)
        right_log = g(lax.rem(ri + 1, N_DEV))
        left_log = g(lax.rem(ri + 7, N_DEV))
        even = lax.rem(ri, 2) == 0
        partner_log = g(lax.rem(jnp.where(even, ri + 3, ri + 5), N_DEV))
        fwd_log = g(lax.rem(jnp.where(even, ri + 7, ri + 1), N_DEV))

        barrier_sem = pltpu.get_barrier_semaphore()
        for nbr in (left_log, right_log, partner_log):
            pl.semaphore_signal(
                barrier_sem, inc=1,
                device_id=(nbr,), device_id_type=pl.DeviceIdType.MESH,
            )
        pl.semaphore_wait(barrier_sem, 3)

        x_bf = x_ref[:, :].astype(jnp.bfloat16)
        gathered[pl.ds(me * m_per, m_per), :] = x_bf

        def mk_ring(direction, h, p):
            if direction == "cw":
                o = g(lax.rem(ri - h + N_DEV, N_DEV))
                dst_dev, sends, recvs = right_log, cw_send, cw_recv
            else:
                o = g(lax.rem(ri + h, N_DEV))
                dst_dev, sends, recvs = left_log, ccw_send, ccw_recv
            sl = pl.ds(o * m_per + p * half, half)
            idx = h * N_PC + p
            return pltpu.make_async_remote_copy(
                src_ref=gathered.at[sl, :],
                dst_ref=gathered.at[sl, :],
                send_sem=sends.at[idx],
                recv_sem=recvs.at[idx],
                device_id=(dst_dev,),
                device_id_type=pl.DeviceIdType.MESH,
            )

        rd = {}
        for p in range(N_PC):
            rd["cw", 0, p] = mk_ring("cw", 0, p)
            rd["cw", 0, p].start()
            rd["ccw", 0, p] = mk_ring("ccw", 0, p)
            rd["ccw", 0, p].start()

        w_bf[:, :] = w_ref[:, :].astype(jnp.bfloat16)

        def gemm(row_start, rows):
            out_ref[pl.ds(row_start, rows), :] = jnp.maximum(
                jnp.dot(
                    gathered[pl.ds(row_start, rows), :],
                    w_bf[:, :],
                    preferred_element_type=jnp.float32,
                ),
                0.0,
            )

        out_ref[pl.ds(me * m_per, m_per), :] = jnp.maximum(
            jnp.dot(x_bf, w_bf[:, :], preferred_element_type=jnp.float32),
            0.0,
        )

        for h in range(N_HOP - 1):
            for p in range(N_PC):
                rd["cw", h, p].wait_recv()
                rd["cw", h + 1, p] = mk_ring("cw", h + 1, p)
                rd["cw", h + 1, p].start()
                rd["ccw", h, p].wait_recv()
                rd["ccw", h + 1, p] = mk_ring("ccw", h + 1, p)
                rd["ccw", h + 1, p].start()
            if h == 0:
                ch_sl = pl.ds(fwd_log * m_per, m_per)
                chord = pltpu.make_async_remote_copy(
                    src_ref=gathered.at[ch_sl, :],
                    dst_ref=gathered.at[ch_sl, :],
                    send_sem=ch_send.at[0],
                    recv_sem=ch_recv.at[0],
                    device_id=(partner_log,),
                    device_id_type=pl.DeviceIdType.MESH,
                )
                chord.start()
            gemm(g(lax.rem(ri + 7 - h, N_DEV)) * m_per, m_per)
            gemm(g(lax.rem(ri + 1 + h, N_DEV)) * m_per, m_per)

        chord.wait_recv()
        gemm(g(lax.rem(ri + 4, N_DEV)) * m_per, m_per)

        o_cw = g(lax.rem(ri + 5, N_DEV))
        o_ccw = g(lax.rem(ri + 3, N_DEV))
        for p in range(N_PC):
            rd["cw", 2, p].wait_recv()
            gemm(o_cw * m_per + p * half, half)
            rd["ccw", 2, p].wait_recv()
            gemm(o_ccw * m_per + p * half, half)

        for h in range(N_HOP):
            for p in range(N_PC):
                rd["cw", h, p].wait_send()
                rd["ccw", h, p].wait_send()
        chord.wait_send()

    return pl.pallas_call(
        body,
        out_shape=jax.ShapeDtypeStruct((N_DEV * m_per, n_per), jnp.float32),
        in_specs=[
            pl.BlockSpec(memory_space=pltpu.VMEM),
            pl.BlockSpec(memory_space=pltpu.VMEM),
        ],
        out_specs=pl.BlockSpec(memory_space=pltpu.VMEM),
        scratch_shapes=[
            pltpu.VMEM((N_DEV * m_per, k), jnp.bfloat16),
            pltpu.VMEM((k, n_per), jnp.bfloat16),
            pltpu.SemaphoreType.DMA((N_HOP * N_PC,)),
            pltpu.SemaphoreType.DMA((N_HOP * N_PC,)),
            pltpu.SemaphoreType.DMA((N_HOP * N_PC,)),
            pltpu.SemaphoreType.DMA((N_HOP * N_PC,)),
            pltpu.SemaphoreType.DMA((1,)),
            pltpu.SemaphoreType.DMA((1,)),
        ],
        compiler_params=pltpu.CompilerParams(collective_id=0),
    )(x, w_mat)


# device time: 43532 ns/iter; 2.4169x vs baseline; 1.0292x over previous
import jax
import jax.numpy as jnp
from jax import lax
from jax.experimental import pallas as pl
from jax.experimental.pallas import tpu as pltpu

N_DEV = 8
N_HOP = 3
N_PC = 4


def kernel(x, w_mat):
    m_per, k = x.shape
    _, n_per = w_mat.shape
    half = m_per // N_PC

    def body(x_ref, w_ref, out_ref, gathered, w_bf, cw_send, cw_recv,
             ccw_send, ccw_recv, ch_send, ch_recv):
        me = lax.axis_index("i")

        def g(r):
            return jnp.where(r < 4, r, 11 - r)

        ri = g(me)
        right_log = g(lax.rem(ri + 1, N_DEV))
        left_log = g(lax.rem(ri + 7, N_DEV))
        even = lax.rem(ri, 2) == 0
        partner_log = g(lax.rem(jnp.where(even, ri + 3, ri + 5), N_DEV))
        fwd_log = g(lax.rem(jnp.where(even, ri + 7, ri + 1), N_DEV))

        barrier_sem = pltpu.get_barrier_semaphore()
        for nbr in (left_log, right_log, partner_log):
            pl.semaphore_signal(
                barrier_sem, inc=1,
                device_id=(nbr,), device_id_type=pl.DeviceIdType.MESH,
            )
        pl.semaphore_wait(barrier_sem, 3)

        my_row0 = me * m_per

        def mk_ring(direction, h, p):
            if direction == "cw":
                o = g(lax.rem(ri - h + N_DEV, N_DEV))
                dst_dev, sends, recvs = right_log, cw_send, cw_recv
            else:
                o = g(lax.rem(ri + h, N_DEV))
                dst_dev, sends, recvs = left_log, ccw_send, ccw_recv
            sl = pl.ds(o * m_per + p * half, half)
            idx = h * N_PC + p
            return pltpu.make_async_remote_copy(
                src_ref=gathered.at[sl, :],
                dst_ref=gathered.at[sl, :],
                send_sem=sends.at[idx],
                recv_sem=recvs.at[idx],
                device_id=(dst_dev,),
                device_id_type=pl.DeviceIdType.MESH,
            )

        rd = {}
        for p in range(N_PC):
            psl = pl.ds(p * half, half)
            gathered[pl.ds(my_row0 + p * half, half), :] = (
                x_ref[psl, :].astype(jnp.bfloat16)
            )
            rd["cw", 0, p] = mk_ring("cw", 0, p)
            rd["cw", 0, p].start()
            rd["ccw", 0, p] = mk_ring("ccw", 0, p)
            rd["ccw", 0, p].start()

        w_bf[:, :] = w_ref[:, :].astype(jnp.bfloat16)

        def gemm(row_start, rows):
            out_ref[pl.ds(row_start, rows), :] = jnp.maximum(
                jnp.dot(
                    gathered[pl.ds(row_start, rows), :],
                    w_bf[:, :],
                    preferred_element_type=jnp.float32,
                ),
                0.0,
            )

        gemm(my_row0, m_per)

        for h in range(N_HOP - 1):
            for p in range(N_PC):
                rd["cw", h, p].wait_recv()
                rd["cw", h + 1, p] = mk_ring("cw", h + 1, p)
                rd["cw", h + 1, p].start()
                rd["ccw", h, p].wait_recv()
                rd["ccw", h + 1, p] = mk_ring("ccw", h + 1, p)
                rd["ccw", h + 1, p].start()
            if h == 0:
                ch_sl = pl.ds(fwd_log * m_per, m_per)
                chord = pltpu.make_async_remote_copy(
                    src_ref=gathered.at[ch_sl, :],
                    dst_ref=gathered.at[ch_sl, :],
                    send_sem=ch_send.at[0],
                    recv_sem=ch_recv.at[0],
                    device_id=(partner_log,),
                    device_id_type=pl.DeviceIdType.MESH,
                )
                chord.start()
            gemm(g(lax.rem(ri + 7 - h, N_DEV)) * m_per, m_per)
            gemm(g(lax.rem(ri + 1 + h, N_DEV)) * m_per, m_per)

        chord.wait_recv()
        gemm(g(lax.rem(ri + 4, N_DEV)) * m_per, m_per)

        o_cw = g(lax.rem(ri + 5, N_DEV))
        o_ccw = g(lax.rem(ri + 3, N_DEV))
        for p in range(N_PC):
            rd["cw", 2, p].wait_recv()
            gemm(o_cw * m_per + p * half, half)
            rd["ccw", 2, p].wait_recv()
            gemm(o_ccw * m_per + p * half, half)

        for h in range(N_HOP):
            for p in range(N_PC):
                rd["cw", h, p].wait_send()
                rd["ccw", h, p].wait_send()
        chord.wait_send()

    return pl.pallas_call(
        body,
        out_shape=jax.ShapeDtypeStruct((N_DEV * m_per, n_per), jnp.float32),
        in_specs=[
            pl.BlockSpec(memory_space=pltpu.VMEM),
            pl.BlockSpec(memory_space=pltpu.VMEM),
        ],
        out_specs=pl.BlockSpec(memory_space=pltpu.VMEM),
        scratch_shapes=[
            pltpu.VMEM((N_DEV * m_per, k), jnp.bfloat16),
            pltpu.VMEM((k, n_per), jnp.bfloat16),
            pltpu.SemaphoreType.DMA((N_HOP * N_PC,)),
            pltpu.SemaphoreType.DMA((N_HOP * N_PC,)),
            pltpu.SemaphoreType.DMA((N_HOP * N_PC,)),
            pltpu.SemaphoreType.DMA((N_HOP * N_PC,)),
            pltpu.SemaphoreType.DMA((1,)),
            pltpu.SemaphoreType.DMA((1,)),
        ],
        compiler_params=pltpu.CompilerParams(collective_id=0),
    )(x, w_mat)
